# baseline (speedup 1.0000x reference)
import functools

import jax
import jax.numpy as jnp
from jax import lax
from jax.experimental import pallas as pl
from jax.experimental.pallas import tpu as pltpu
from jax.experimental.pallas import tpu_sc as plsc


def _sc_gather_t(tab_t, xs_t):
    f_dim, e_dim, v_dim = tab_t.shape
    b_dim = xs_t.shape[1]
    info = plsc.get_sparse_core_info()
    nc, ns = info.num_cores, info.num_subcores
    nw = nc * ns
    rows_total = f_dim * e_dim
    rpw = rows_total // nw
    ic = 4096
    icw = ic // 2
    n_ic = b_dim // ic
    assert rows_total % nw == 0 and b_dim % ic == 0

    mesh = plsc.VectorSubcoreMesh(core_axis_name="c", subcore_axis_name="s")

    @functools.partial(
        pl.kernel,
        mesh=mesh,
        out_type=jax.ShapeDtypeStruct((f_dim, e_dim, b_dim // 2), jnp.int32),
        compiler_params=pltpu.CompilerParams(needs_layout_passes=False),
        scratch_types=[
            pltpu.VMEM((v_dim,), jnp.float32),
            pltpu.VMEM((b_dim,), jnp.int32),
            pltpu.VMEM((icw,), jnp.int32),
            pltpu.VMEM((icw,), jnp.int32),
            pltpu.SemaphoreType.DMA,
            pltpu.SemaphoreType.DMA,
            pltpu.SemaphoreType.DMA,
        ],
    )
    def gk(tab_hbm, idx_hbm, out_hbm, rowbuf, idxbuf, ob0, ob1, semr, so0, so1):
        wid = lax.axis_index("s") * nc + lax.axis_index("c")
        obufs, osems = (ob0, ob1), (so0, so1)
        iota2 = lax.iota(jnp.int32, 16) * 2

        def row_body(r, f_last):
            g = wid * rpw + r
            f = g // e_dim
            e = g % e_dim

            @pl.when(f != f_last)
            def _():
                pltpu.sync_copy(idx_hbm.at[f, :], idxbuf)

            pltpu.async_copy(tab_hbm.at[f, e, :], rowbuf, semr).wait()
            for c in range(n_ic):
                ob, osem = obufs[c % 2], osems[c % 2]

                def drain(ob=ob, osem=osem):
                    pltpu.make_async_copy(
                        out_hbm.at[f, e, pl.ds(c * icw, icw)], ob, osem).wait()
                if c >= 2:
                    drain()
                else:
                    pl.when(r >= 1)(drain)

                @plsc.parallel_loop(0, ic // 32, unroll=8)
                def _(i):
                    ev = iota2 + (c * ic + i * 32)
                    iv_a = plsc.load_gather(idxbuf, [ev])
                    iv_b = plsc.load_gather(idxbuf, [ev + 1])
                    va = plsc.load_gather(rowbuf, [iv_a])
                    vb = plsc.load_gather(rowbuf, [iv_b])
                    pair = plsc.pack(va, vb,
                                     format=plsc.PackFormat.INTERLEAVED,
                                     preferred_element_type=jnp.bfloat16)
                    ob[pl.ds(i * 16, 16)] = plsc.bitcast(pair, jnp.int32)

                pltpu.async_copy(ob, out_hbm.at[f, e, pl.ds(c * icw, icw)],
                                 osem)
            return f

        f_fin = lax.fori_loop(0, rpw, row_body, jnp.int32(-1))
        g = wid * rpw + rpw - 1
        f, e = g // e_dim, g % e_dim
        for c in (n_ic - 2, n_ic - 1):
            pltpu.make_async_copy(
                out_hbm.at[f, e, pl.ds(c * icw, icw)],
                obufs[c % 2], osems[c % 2]).wait()
        del f_fin

    return gk(tab_t, xs_t)


def _mlp_body(e_ref, d_ref, w1a, w1b, b1, w2, b2, w3, b3, wo, bo, o_ref):
    ct0 = (((0,), (0,)), ((), ()))
    x = lax.dot_general(e_ref[...], w1a[...], ct0,
                        preferred_element_type=jnp.float32)
    x = x + lax.dot_general(d_ref[...], w1b[...], ct0,
                            preferred_element_type=jnp.float32)
    h = jnp.maximum(x + b1[...], 0.0)
    h = jnp.maximum(jnp.dot(h, w2[...], preferred_element_type=jnp.float32)
                    + b2[...], 0.0)
    h = jnp.maximum(jnp.dot(h, w3[...], preferred_element_type=jnp.float32)
                    + b3[...], 0.0)
    o = lax.dot_general(wo[...], h, (((0,), (1,)), ((), ())),
                        preferred_element_type=jnp.float32)
    o_ref[...] = jax.nn.sigmoid(o + bo[...])


def _mlp(emb_t2, xd_t, w1a, w1b, b1, w2, b2, w3, b3, wout, bout,
         interpret=False):
    de, b = emb_t2.shape
    dd = xd_t.shape[0]
    tb = 512
    h1, h2, h3 = w2.shape[0], w3.shape[0], wout.shape[0]
    full = lambda shape: pl.BlockSpec(shape, lambda i: tuple(0 for _ in shape))
    return pl.pallas_call(
        _mlp_body,
        grid=(b // tb,),
        in_specs=[
            pl.BlockSpec((de, tb), lambda i: (0, i)),
            pl.BlockSpec((dd, tb), lambda i: (0, i)),
            full((de, h1)), full((dd, h1)), full((1, h1)),
            full((h1, h2)), full((1, h2)),
            full((h2, h3)), full((1, h3)),
            full((h3, 1)), full((1, 1)),
        ],
        out_specs=pl.BlockSpec((1, tb), lambda i: (0, i)),
        out_shape=jax.ShapeDtypeStruct((1, b), jnp.float32),
        interpret=interpret,
    )(emb_t2, xd_t, w1a, w1b, b1.reshape(1, -1), w2, b2.reshape(1, -1),
      w3, b3.reshape(1, -1), wout, bout.reshape(1, 1))


def kernel(X_sparse, X_dense, tables, W1, b1, W2, b2, W3, b3, Wout, bout):
    b, f = X_sparse.shape
    v, e = tables.shape[1], tables.shape[2]
    tab_t = tables.transpose(0, 2, 1)
    xs_t = X_sparse.T
    emb_w = _sc_gather_t(tab_t, xs_t)
    emb_bf = lax.bitcast_convert_type(emb_w, jnp.bfloat16)
    emb_t2 = emb_bf.reshape(f * e, b)
    w1a = W1[: f * e].astype(jnp.bfloat16)
    w1b = W1[f * e:]
    out = _mlp(emb_t2, X_dense.T, w1a, w1b, b1, W2, b2, W3, b3, Wout, bout)
    return out.T


# speedup vs baseline: 2.7615x; 2.7615x over previous
import functools

import jax
import jax.numpy as jnp
from jax import lax
from jax.experimental import pallas as pl
from jax.experimental.pallas import tpu as pltpu
from jax.experimental.pallas import tpu_sc as plsc


def _sc_gather_t(tab_t, xs_t):
    f_dim, e_dim, v_dim = tab_t.shape
    b_dim = xs_t.shape[1]
    info = plsc.get_sparse_core_info()
    nc, ns = info.num_cores, info.num_subcores
    nw = nc * ns
    rows_total = f_dim * e_dim
    rpw = rows_total // nw
    ic = 4096
    icw = ic // 2
    n_ic = b_dim // ic
    assert rows_total % nw == 0 and b_dim % ic == 0

    mesh = plsc.VectorSubcoreMesh(core_axis_name="c", subcore_axis_name="s")

    @functools.partial(
        pl.kernel,
        mesh=mesh,
        out_type=jax.ShapeDtypeStruct((f_dim, e_dim, b_dim), jnp.float32),
        compiler_params=pltpu.CompilerParams(needs_layout_passes=False),
        scratch_types=[
            pltpu.VMEM((v_dim,), jnp.float32),
            pltpu.VMEM((b_dim,), jnp.int32),
            pltpu.VMEM((ic,), jnp.float32),
            pltpu.VMEM((ic,), jnp.float32),
            pltpu.SemaphoreType.DMA,
            pltpu.SemaphoreType.DMA,
            pltpu.SemaphoreType.DMA,
        ],
    )
    def gk(tab_hbm, idx_hbm, out_hbm, rowbuf, idxbuf, ob0, ob1, semr, so0, so1):
        wid = lax.axis_index("s") * nc + lax.axis_index("c")
        obufs, osems = (ob0, ob1), (so0, so1)

        def row_body(r, f_last):
            g = wid * rpw + r
            f = g // e_dim
            e = g % e_dim

            @pl.when(f != f_last)
            def _():
                pltpu.sync_copy(idx_hbm.at[f, :], idxbuf)

            pltpu.async_copy(tab_hbm.at[f, e, :], rowbuf, semr).wait()
            for c in range(n_ic):
                ob, osem = obufs[c % 2], osems[c % 2]

                def drain(ob=ob, osem=osem):
                    pltpu.make_async_copy(
                        out_hbm.at[f, e, pl.ds(c * ic, ic)], ob, osem).wait()
                if c >= 2:
                    drain()
                else:
                    pl.when(r >= 1)(drain)

                @plsc.parallel_loop(0, ic // 16, unroll=8)
                def _(i):
                    iv = idxbuf[pl.ds(c * ic + i * 16, 16)]
                    ob[pl.ds(i * 16, 16)] = plsc.load_gather(rowbuf, [iv])

                pltpu.async_copy(ob, out_hbm.at[f, e, pl.ds(c * ic, ic)],
                                 osem)
            return f

        f_fin = lax.fori_loop(0, rpw, row_body, jnp.int32(-1))
        g = wid * rpw + rpw - 1
        f, e = g // e_dim, g % e_dim
        for c in (n_ic - 2, n_ic - 1):
            pltpu.make_async_copy(
                out_hbm.at[f, e, pl.ds(c * ic, ic)],
                obufs[c % 2], osems[c % 2]).wait()
        del f_fin

    return gk(tab_t, xs_t)


def _mlp_body(e_ref, d_ref, w1a, w1b, b1, w2, b2, w3, b3, wo, bo, o_ref):
    ct0 = (((0,), (0,)), ((), ()))
    x = lax.dot_general(e_ref[...].astype(jnp.bfloat16), w1a[...], ct0,
                        preferred_element_type=jnp.float32)
    x = x + lax.dot_general(d_ref[...], w1b[...], ct0,
                            preferred_element_type=jnp.float32)
    h = jnp.maximum(x + b1[...], 0.0)
    h = jnp.maximum(jnp.dot(h, w2[...], preferred_element_type=jnp.float32)
                    + b2[...], 0.0)
    h = jnp.maximum(jnp.dot(h, w3[...], preferred_element_type=jnp.float32)
                    + b3[...], 0.0)
    o = lax.dot_general(wo[...], h, (((0,), (1,)), ((), ())),
                        preferred_element_type=jnp.float32)
    o_ref[...] = jax.nn.sigmoid(o + bo[...])


def _mlp(emb_t2, xd_t, w1a, w1b, b1, w2, b2, w3, b3, wout, bout,
         interpret=False):
    de, b = emb_t2.shape
    dd = xd_t.shape[0]
    tb = 512
    h1, h2, h3 = w2.shape[0], w3.shape[0], wout.shape[0]
    full = lambda shape: pl.BlockSpec(shape, lambda i: tuple(0 for _ in shape))
    return pl.pallas_call(
        _mlp_body,
        grid=(b // tb,),
        in_specs=[
            pl.BlockSpec((de, tb), lambda i: (0, i)),
            pl.BlockSpec((dd, tb), lambda i: (0, i)),
            full((de, h1)), full((dd, h1)), full((1, h1)),
            full((h1, h2)), full((1, h2)),
            full((h2, h3)), full((1, h3)),
            full((h3, 1)), full((1, 1)),
        ],
        out_specs=pl.BlockSpec((1, tb), lambda i: (0, i)),
        out_shape=jax.ShapeDtypeStruct((1, b), jnp.float32),
        interpret=interpret,
    )(emb_t2, xd_t, w1a, w1b, b1.reshape(1, -1), w2, b2.reshape(1, -1),
      w3, b3.reshape(1, -1), wout, bout.reshape(1, 1))


def kernel(X_sparse, X_dense, tables, W1, b1, W2, b2, W3, b3, Wout, bout):
    b, f = X_sparse.shape
    v, e = tables.shape[1], tables.shape[2]
    tab_t = tables.transpose(0, 2, 1)
    xs_t = X_sparse.T
    emb_t = _sc_gather_t(tab_t, xs_t)
    emb_t2 = emb_t.reshape(f * e, b)
    w1a = W1[: f * e].astype(jnp.bfloat16)
    w1b = W1[f * e:]
    out = _mlp(emb_t2, X_dense.T, w1a, w1b, b1, W2, b2, W3, b3, Wout, bout)
    return out.T


# speedup vs baseline: 2.8698x; 1.0392x over previous
import functools

import jax
import jax.numpy as jnp
from jax import lax
from jax.experimental import pallas as pl
from jax.experimental.pallas import tpu as pltpu
from jax.experimental.pallas import tpu_sc as plsc


def _sc_gather_t(tab_t, xs_t):
    f_dim, e_dim, v_dim = tab_t.shape
    b_dim = xs_t.shape[1]
    info = plsc.get_sparse_core_info()
    nc, ns = info.num_cores, info.num_subcores
    nw = nc * ns
    rows_total = f_dim * e_dim
    rpw = rows_total // nw
    ic = 4096
    icw = ic // 2
    n_ic = b_dim // ic
    assert rows_total % nw == 0 and b_dim % ic == 0

    mesh = plsc.VectorSubcoreMesh(core_axis_name="c", subcore_axis_name="s")

    @functools.partial(
        pl.kernel,
        mesh=mesh,
        out_type=jax.ShapeDtypeStruct((f_dim, e_dim, b_dim), jnp.float32),
        compiler_params=pltpu.CompilerParams(needs_layout_passes=False),
        scratch_types=[
            pltpu.VMEM((v_dim,), jnp.float32),
            pltpu.VMEM((b_dim,), jnp.int32),
            pltpu.VMEM((ic,), jnp.float32),
            pltpu.VMEM((ic,), jnp.float32),
            pltpu.SemaphoreType.DMA,
            pltpu.SemaphoreType.DMA,
            pltpu.SemaphoreType.DMA,
        ],
    )
    def gk(tab_hbm, idx_hbm, out_hbm, rowbuf, idxbuf, ob0, ob1, semr, so0, so1):
        wid = lax.axis_index("s") * nc + lax.axis_index("c")
        obufs, osems = (ob0, ob1), (so0, so1)

        def row_body(r, f_last):
            g = wid * rpw + r
            f = g // e_dim
            e = g % e_dim

            @pl.when(f != f_last)
            def _():
                pltpu.sync_copy(idx_hbm.at[f, :], idxbuf)

            pltpu.async_copy(tab_hbm.at[f, e, :], rowbuf, semr).wait()
            for c in range(n_ic):
                ob, osem = obufs[c % 2], osems[c % 2]

                def drain(ob=ob, osem=osem):
                    pltpu.make_async_copy(
                        out_hbm.at[f, e, pl.ds(c * ic, ic)], ob, osem).wait()
                if c >= 2:
                    drain()
                else:
                    pl.when(r >= 1)(drain)

                @plsc.parallel_loop(0, ic // 16, unroll=8)
                def _(i):
                    iv = idxbuf[pl.ds(c * ic + i * 16, 16)]
                    ob[pl.ds(i * 16, 16)] = plsc.load_gather(rowbuf, [iv])

                pltpu.async_copy(ob, out_hbm.at[f, e, pl.ds(c * ic, ic)],
                                 osem)
            return f

        f_fin = lax.fori_loop(0, rpw, row_body, jnp.int32(-1))
        g = wid * rpw + rpw - 1
        f, e = g // e_dim, g % e_dim
        for c in (n_ic - 2, n_ic - 1):
            pltpu.make_async_copy(
                out_hbm.at[f, e, pl.ds(c * ic, ic)],
                obufs[c % 2], osems[c % 2]).wait()
        del f_fin

    return gk(tab_t, xs_t)


def _mlp_body(e_ref, d_ref, w1a, w1b, b1, w2, b2, w3, b3, wo, bo, o_ref):
    ct0 = (((0,), (0,)), ((), ()))
    x = lax.dot_general(e_ref[...].astype(jnp.bfloat16), w1a[...], ct0,
                        preferred_element_type=jnp.float32)
    x = x + lax.dot_general(d_ref[...], w1b[...], ct0,
                            preferred_element_type=jnp.float32)
    h = jnp.maximum(x + b1[...], 0.0)
    h = jnp.maximum(jnp.dot(h, w2[...], preferred_element_type=jnp.float32)
                    + b2[...], 0.0)
    h = jnp.maximum(jnp.dot(h, w3[...], preferred_element_type=jnp.float32)
                    + b3[...], 0.0)
    o = lax.dot_general(wo[...], h, (((0,), (1,)), ((), ())),
                        preferred_element_type=jnp.float32)
    o_ref[...] = jax.nn.sigmoid(o + bo[...])


def _mlp(emb_t2, xd_t, w1a, w1b, b1, w2, b2, w3, b3, wout, bout,
         interpret=False):
    de, b = emb_t2.shape
    dd = xd_t.shape[0]
    tb = 1024
    h1, h2, h3 = w2.shape[0], w3.shape[0], wout.shape[0]
    full = lambda shape: pl.BlockSpec(shape, lambda i: tuple(0 for _ in shape))
    return pl.pallas_call(
        _mlp_body,
        grid=(b // tb,),
        in_specs=[
            pl.BlockSpec((de, tb), lambda i: (0, i)),
            pl.BlockSpec((dd, tb), lambda i: (0, i)),
            full((de, h1)), full((dd, h1)), full((1, h1)),
            full((h1, h2)), full((1, h2)),
            full((h2, h3)), full((1, h3)),
            full((h3, 1)), full((1, 1)),
        ],
        out_specs=pl.BlockSpec((1, tb), lambda i: (0, i)),
        out_shape=jax.ShapeDtypeStruct((1, b), jnp.float32),
        interpret=interpret,
    )(emb_t2, xd_t, w1a, w1b, b1.reshape(1, -1), w2, b2.reshape(1, -1),
      w3, b3.reshape(1, -1), wout, bout.reshape(1, 1))


def kernel(X_sparse, X_dense, tables, W1, b1, W2, b2, W3, b3, Wout, bout):
    b, f = X_sparse.shape
    v, e = tables.shape[1], tables.shape[2]
    tab_t = tables.transpose(0, 2, 1)
    xs_t = X_sparse.T
    emb_t = _sc_gather_t(tab_t, xs_t)
    emb_t2 = emb_t.reshape(f * e, b)
    w1a = W1[: f * e].astype(jnp.bfloat16)
    w1b = W1[f * e:]
    out = _mlp(emb_t2, X_dense.T, w1a, w1b, b1, W2, b2, W3, b3, Wout, bout)
    return out.T


# speedup vs baseline: 2.9036x; 1.0118x over previous
import functools

import jax
import jax.numpy as jnp
from jax import lax
from jax.experimental import pallas as pl
from jax.experimental.pallas import tpu as pltpu
from jax.experimental.pallas import tpu_sc as plsc


def _sc_gather_t(tab_t, xs_t):
    f_dim, e_dim, v_dim = tab_t.shape
    b_dim = xs_t.shape[1]
    info = plsc.get_sparse_core_info()
    nc, ns = info.num_cores, info.num_subcores
    nw = nc * ns
    rows_total = f_dim * e_dim
    rpw = rows_total // nw
    ic = 4096
    icw = ic // 2
    n_ic = b_dim // ic
    assert rows_total % nw == 0 and b_dim % ic == 0

    mesh = plsc.VectorSubcoreMesh(core_axis_name="c", subcore_axis_name="s")

    @functools.partial(
        pl.kernel,
        mesh=mesh,
        out_type=jax.ShapeDtypeStruct((f_dim, e_dim, b_dim), jnp.float32),
        compiler_params=pltpu.CompilerParams(needs_layout_passes=False),
        scratch_types=[
            pltpu.VMEM((v_dim,), jnp.float32),
            pltpu.VMEM((b_dim,), jnp.int32),
            pltpu.VMEM((ic,), jnp.float32),
            pltpu.VMEM((ic,), jnp.float32),
            pltpu.SemaphoreType.DMA,
            pltpu.SemaphoreType.DMA,
            pltpu.SemaphoreType.DMA,
        ],
    )
    def gk(tab_hbm, idx_hbm, out_hbm, rowbuf, idxbuf, ob0, ob1, semr, so0, so1):
        wid = lax.axis_index("s") * nc + lax.axis_index("c")
        obufs, osems = (ob0, ob1), (so0, so1)

        def row_body(r, f_last):
            g = wid * rpw + r
            f = g // e_dim
            e = g % e_dim

            @pl.when(f != f_last)
            def _():
                pltpu.sync_copy(idx_hbm.at[f, :], idxbuf)

            pltpu.async_copy(tab_hbm.at[f, e, :], rowbuf, semr).wait()
            for c in range(n_ic):
                ob, osem = obufs[c % 2], osems[c % 2]

                def drain(ob=ob, osem=osem):
                    pltpu.make_async_copy(
                        out_hbm.at[f, e, pl.ds(c * ic, ic)], ob, osem).wait()
                if c >= 2:
                    drain()
                else:
                    pl.when(r >= 1)(drain)

                @plsc.parallel_loop(0, ic // 16, unroll=8)
                def _(i):
                    iv = idxbuf[pl.ds(c * ic + i * 16, 16)]
                    ob[pl.ds(i * 16, 16)] = plsc.load_gather(rowbuf, [iv])

                pltpu.async_copy(ob, out_hbm.at[f, e, pl.ds(c * ic, ic)],
                                 osem)
            return f

        f_fin = lax.fori_loop(0, rpw, row_body, jnp.int32(-1))
        g = wid * rpw + rpw - 1
        f, e = g // e_dim, g % e_dim
        for c in (n_ic - 2, n_ic - 1):
            pltpu.make_async_copy(
                out_hbm.at[f, e, pl.ds(c * ic, ic)],
                obufs[c % 2], osems[c % 2]).wait()
        del f_fin

    return gk(tab_t, xs_t)


def _mlp_body(e_ref, d_ref, w1a, w1b, b1, w2, b2, w3, b3, wo, bo, o_ref):
    ct0 = (((0,), (0,)), ((), ()))
    x = lax.dot_general(e_ref[...].astype(jnp.bfloat16), w1a[...], ct0,
                        preferred_element_type=jnp.float32)
    x = x + lax.dot_general(d_ref[...], w1b[...], ct0,
                            preferred_element_type=jnp.float32)
    h = jnp.maximum(x + b1[...], 0.0)
    h = jnp.maximum(jnp.dot(h, w2[...], preferred_element_type=jnp.float32)
                    + b2[...], 0.0)
    h = jnp.maximum(jnp.dot(h, w3[...], preferred_element_type=jnp.float32)
                    + b3[...], 0.0)
    o = lax.dot_general(wo[...], h, (((0,), (1,)), ((), ())),
                        preferred_element_type=jnp.float32)
    o_ref[...] = jax.nn.sigmoid(o + bo[...])


def _mlp(emb_t2, xd_t, w1a, w1b, b1, w2, b2, w3, b3, wout, bout,
         interpret=False):
    de, b = emb_t2.shape
    dd = xd_t.shape[0]
    tb = 2048
    h1, h2, h3 = w2.shape[0], w3.shape[0], wout.shape[0]
    full = lambda shape: pl.BlockSpec(shape, lambda i: tuple(0 for _ in shape))
    return pl.pallas_call(
        _mlp_body,
        grid=(b // tb,),
        in_specs=[
            pl.BlockSpec((de, tb), lambda i: (0, i)),
            pl.BlockSpec((dd, tb), lambda i: (0, i)),
            full((de, h1)), full((dd, h1)), full((1, h1)),
            full((h1, h2)), full((1, h2)),
            full((h2, h3)), full((1, h3)),
            full((h3, 1)), full((1, 1)),
        ],
        out_specs=pl.BlockSpec((1, tb), lambda i: (0, i)),
        out_shape=jax.ShapeDtypeStruct((1, b), jnp.float32),
        interpret=interpret,
    )(emb_t2, xd_t, w1a, w1b, b1.reshape(1, -1), w2, b2.reshape(1, -1),
      w3, b3.reshape(1, -1), wout, bout.reshape(1, 1))


def kernel(X_sparse, X_dense, tables, W1, b1, W2, b2, W3, b3, Wout, bout):
    b, f = X_sparse.shape
    v, e = tables.shape[1], tables.shape[2]
    tab_t = tables.transpose(0, 2, 1)
    xs_t = X_sparse.T
    emb_t = _sc_gather_t(tab_t, xs_t)
    emb_t2 = emb_t.reshape(f * e, b)
    w1a = W1[: f * e].astype(jnp.bfloat16)
    w1b = W1[f * e:]
    out = _mlp(emb_t2, X_dense.T, w1a, w1b, b1, W2, b2, W3, b3, Wout, bout)
    return out.T
